# baseline (speedup 1.0000x reference)
import functools

import jax
import jax.numpy as jnp
from jax import lax
from jax.experimental import pallas as pl
from jax.experimental.pallas import tpu as pltpu
from jax.experimental.pallas import tpu_sc as plsc

N = 10000
D_IN = 128
H = 128
D_OUT = 64
E = 320000

NPAD = 10240
NW = 32
CB = 128
NCH0 = 58
NCH1 = 102
NCHMAX = max(NCH0, NCH1)
EPAD = 16 * (NCH0 + NCH1) * CB
RPT = NPAD // 16

_MESH = plsc.VectorSubcoreMesh(core_axis_name="c", subcore_axis_name="s")
_BN = float(1.0 / (1.0 + 1e-5) ** 0.5)


def _lrelu(t):
    return jnp.where(t >= 0, t, 0.2 * t)


@functools.partial(
    pl.kernel,
    out_type=jax.ShapeDtypeStruct((2, NPAD), jnp.float32),
    mesh=_MESH,
    scratch_types=[
        pltpu.VMEM((NCHMAX, CB), jnp.int32),
        pltpu.VMEM((CB,), jnp.float32),
        pltpu.VMEM_SHARED((NPAD,), jnp.float32),
    ],
    name="deg",
)
def _deg_kernel(dst_hbm, zeros_hbm, out_hbm, dst_v, ones_v, deg_sh):
    cid = lax.axis_index("c")
    sid = lax.axis_index("s")
    wid = cid * 16 + sid
    nch = jnp.where(cid == 0, NCH0, NCH1)
    base = pl.multiple_of(sid * RPT, 8)
    pltpu.sync_copy(zeros_hbm.at[pl.ds(base, RPT)], deg_sh.at[pl.ds(base, RPT)])
    pltpu.sync_copy(dst_hbm.at[wid], dst_v)
    for i in range(CB // 16):
        ones_v[pl.ds(i * 16, 16)] = jnp.ones((16,), jnp.float32)
    plsc.subcore_barrier()

    def body(j, carry):
        pltpu.sync_copy(ones_v, deg_sh.at[dst_v.at[j]], add=True)
        return carry

    lax.fori_loop(0, nch, body, 0)
    plsc.subcore_barrier()
    pltpu.sync_copy(deg_sh.at[pl.ds(base, RPT)], out_hbm.at[cid, pl.ds(base, RPT)])


def _make_agg(hdim):
    @functools.partial(
        pl.kernel,
        out_type=jax.ShapeDtypeStruct((2, NPAD, hdim), jnp.float32),
        mesh=_MESH,
        scratch_types=[
            pltpu.VMEM((NCHMAX, CB), jnp.int32),
            pltpu.VMEM((NCHMAX, CB), jnp.int32),
            pltpu.VMEM((CB, hdim), jnp.float32),
            pltpu.VMEM_SHARED((NPAD, hdim), jnp.float32),
            pltpu.SemaphoreType.DMA,
        ],
        name="agg%d" % hdim,
        compiler_params=pltpu.CompilerParams(use_tc_tiling_on_sc=False),
    )
    def _agg(src_hbm, dst_hbm, z_hbm, out_hbm, src_v, dst_v, rows_v, acc_sh, sem):
        cid = lax.axis_index("c")
        sid = lax.axis_index("s")
        wid = cid * 16 + sid
        nch = jnp.where(cid == 0, NCH0, NCH1)
        base = pl.multiple_of(sid * RPT, 8)
        pltpu.sync_copy(z_hbm.at[pl.ds(base, RPT)], acc_sh.at[pl.ds(base, RPT)])
        pltpu.sync_copy(src_hbm.at[wid], src_v)
        pltpu.sync_copy(dst_hbm.at[wid], dst_v)
        plsc.subcore_barrier()

        def body(j, carry):
            pltpu.async_copy(z_hbm.at[src_v.at[j]], rows_v, sem).wait()
            pltpu.sync_copy(rows_v, acc_sh.at[dst_v.at[j]], add=True)
            return carry

        lax.fori_loop(0, nch, body, 0)
        plsc.subcore_barrier()
        pltpu.sync_copy(acc_sh.at[pl.ds(base, RPT)],
                        out_hbm.at[cid, pl.ds(base, RPT)])

    return _agg


_agg128 = _make_agg(H)
_agg64 = _make_agg(D_OUT)


def _tc_pre_body(x_ref, deg_ref, wp_ref, bp_ref, w0_ref, h0_ref, z0_ref, dinv_ref):
    d = deg_ref[...]
    dinv = lax.rsqrt(d[:, 0:1] + d[:, 1:2] + 1.0)
    h0 = _lrelu(jnp.dot(x_ref[...], wp_ref[...],
                        preferred_element_type=jnp.float32) + bp_ref[...])
    h0_ref[...] = h0
    z0_ref[...] = jnp.dot(h0, w0_ref[...],
                          preferred_element_type=jnp.float32) * dinv
    dinv_ref[...] = dinv


_tc_pre = pl.pallas_call(
    _tc_pre_body,
    out_shape=(
        jax.ShapeDtypeStruct((NPAD, H), jnp.float32),
        jax.ShapeDtypeStruct((NPAD, H), jnp.float32),
        jax.ShapeDtypeStruct((NPAD, 1), jnp.float32),
    ),
)


def _make_tc_mid(dout):
    def body(acc_ref, z_ref, h_ref, dinv_ref, b_ref, g_ref, be_ref, w_ref,
             hn_ref, zn_ref):
        dinv = dinv_ref[...]
        s = (acc_ref[0] + acc_ref[1] - z_ref[...]) * dinv + b_ref[...]
        s = s * g_ref[...] + be_ref[...]
        hn = _lrelu(s) + h_ref[...]
        hn_ref[...] = hn
        zn_ref[...] = jnp.dot(hn, w_ref[...],
                              preferred_element_type=jnp.float32) * dinv

    return pl.pallas_call(
        body,
        out_shape=(
            jax.ShapeDtypeStruct((NPAD, H), jnp.float32),
            jax.ShapeDtypeStruct((NPAD, dout), jnp.float32),
        ),
    )


_tc_mid128 = _make_tc_mid(H)
_tc_mid64 = _make_tc_mid(D_OUT)


def _tc_fin_body(acc_ref, z_ref, dinv_ref, b2_ref, a1_ref, ab1_ref, a2_ref,
                 ab2_ref, wc_ref, out_ref):
    h3p = (acc_ref[0] + acc_ref[1] - z_ref[...]) * dinv_ref[...] + b2_ref[...]
    h3 = h3p[:N]
    a = _lrelu(jnp.dot(h3, a1_ref[...],
                       preferred_element_type=jnp.float32) + ab1_ref[...])
    s = jnp.dot(a, a2_ref[...], preferred_element_type=jnp.float32) + ab2_ref[...]
    m = jnp.max(s, axis=0, keepdims=True)
    e = jnp.exp(s - m)
    sm = e / jnp.sum(e, axis=0, keepdims=True)
    logit = jnp.sum(sm * wc_ref[...], axis=1, keepdims=True)
    cw = 1.0 / (1.0 + jnp.exp(-logit))
    out_ref[...] = h3 * cw


_tc_fin = pl.pallas_call(
    _tc_fin_body,
    out_shape=jax.ShapeDtypeStruct((N, D_OUT), jnp.float32),
)


def _split_chunks(flat, fill):
    chunks = flat.reshape(-1, CB)
    p0 = chunks[:16 * NCH0].reshape(16, NCH0, CB)
    p1 = chunks[16 * NCH0:].reshape(16, NCH1, CB)
    p0 = jnp.pad(p0, ((0, 0), (0, NCHMAX - NCH0), (0, 0)),
                 constant_values=fill)
    p1 = jnp.pad(p1, ((0, 0), (0, NCHMAX - NCH1), (0, 0)),
                 constant_values=fill)
    return jnp.concatenate([p0, p1], axis=0)


def kernel(x, edge_index, Wp, bp, W0, b0, W1, b1, W2, b2, g0, be0, g1, be1,
           Aw1, Ab1, Aw2, Ab2, Wc):
    f32 = jnp.float32
    src = edge_index[0].astype(jnp.int32)
    dst = edge_index[1].astype(jnp.int32)
    pad = EPAD - E
    src3 = _split_chunks(
        jnp.concatenate([src, jnp.zeros((pad,), jnp.int32)]), 0)
    dst3 = _split_chunks(
        jnp.concatenate([dst, jnp.full((pad,), N, jnp.int32)]), N)
    zerosN = jnp.zeros((NPAD,), f32)
    xpad = jnp.pad(x, ((0, NPAD - N), (0, 0)))

    deg = _deg_kernel(dst3, zerosN)
    degT = deg.T

    h0, z0, dinv = _tc_pre(xpad, degT, Wp, bp.reshape(1, -1), W0)
    acc0 = _agg128(src3, dst3, z0)
    h1, z1 = _tc_mid128(acc0, z0, h0, dinv, b0.reshape(1, -1),
                        (g0 * _BN).reshape(1, -1), be0.reshape(1, -1), W1)
    acc1 = _agg128(src3, dst3, z1)
    h2, z2 = _tc_mid64(acc1, z1, h1, dinv, b1.reshape(1, -1),
                       (g1 * _BN).reshape(1, -1), be1.reshape(1, -1), W2)
    del h2
    acc2 = _agg64(src3, dst3, z2)

    a1cat = jnp.transpose(Aw1, (1, 0, 2)).reshape(D_OUT, D_OUT)
    ab1cat = Ab1.reshape(1, D_OUT)
    a2bd = jax.scipy.linalg.block_diag(*[Aw2[i] for i in range(4)])
    ab2cat = Ab2.reshape(1, 4)
    wcr = Wc.reshape(1, 4)

    return _tc_fin(acc2, z2, dinv, b2.reshape(1, -1), a1cat, ab1cat,
                   a2bd, ab2cat, wcr)


# speedup vs baseline: 3.6102x; 3.6102x over previous
import functools

import jax
import jax.numpy as jnp
from jax import lax
from jax.experimental import pallas as pl
from jax.experimental.pallas import tpu as pltpu
from jax.experimental.pallas import tpu_sc as plsc

N = 10000
D_IN = 128
H = 128
D_OUT = 64
E = 320000

NPAD = 10240
NW = 32
CB = 100
NCH = 100
BLK = 25
EPAD = NW * NCH * CB
RPT = NPAD // 16

_MESH = plsc.VectorSubcoreMesh(core_axis_name="c", subcore_axis_name="s")
_BN = float(1.0 / (1.0 + 1e-5) ** 0.5)


def _lrelu(t):
    return jnp.where(t >= 0, t, 0.2 * t)


@functools.partial(
    pl.kernel,
    out_type=jax.ShapeDtypeStruct((2, NPAD), jnp.float32),
    mesh=_MESH,
    scratch_types=[
        pltpu.VMEM((NCH, CB), jnp.int32),
        pltpu.VMEM((16 * (-(-CB // 16)),), jnp.float32),
        pltpu.VMEM_SHARED((NPAD,), jnp.float32),
    ],
    name="deg",
)
def _deg_kernel(dst_hbm, zeros_hbm, out_hbm, dst_v, ones_v, deg_sh):
    cid = lax.axis_index("c")
    sid = lax.axis_index("s")
    wid = cid * 16 + sid
    base = pl.multiple_of(sid * RPT, 8)
    pltpu.sync_copy(zeros_hbm.at[pl.ds(base, RPT)], deg_sh.at[pl.ds(base, RPT)])
    pltpu.sync_copy(dst_hbm.at[wid], dst_v)
    for i in range(-(-CB // 16)):
        ones_v[pl.ds(i * 16, 16)] = jnp.ones((16,), jnp.float32)
    plsc.subcore_barrier()

    def body(j, carry):
        pltpu.sync_copy(ones_v.at[pl.ds(0, CB)], deg_sh.at[dst_v.at[j]],
                        add=True)
        return carry

    lax.fori_loop(0, NCH, body, 0)
    plsc.subcore_barrier()
    pltpu.sync_copy(deg_sh.at[pl.ds(base, RPT)], out_hbm.at[cid, pl.ds(base, RPT)])


def _make_agg(hdim):
    @functools.partial(
        pl.kernel,
        out_type=jax.ShapeDtypeStruct((2, NPAD, hdim), jnp.float32),
        mesh=_MESH,
        scratch_types=[
            pltpu.VMEM((NCH, CB), jnp.int32),
            pltpu.VMEM((NCH, CB), jnp.int32),
            pltpu.VMEM((CB, hdim), jnp.float32),
            pltpu.VMEM((CB, hdim), jnp.float32),
            pltpu.VMEM_SHARED((NPAD, hdim), jnp.float32),
            pltpu.SemaphoreType.DMA,
            pltpu.SemaphoreType.DMA,
        ],
        name="agg%d" % hdim,
        compiler_params=pltpu.CompilerParams(use_tc_tiling_on_sc=False),
    )
    def _agg(src_hbm, dst_hbm, z_hbm, out_hbm, src_v, dst_v, r0, r1, acc_sh,
             sem0, sem1):
        rows = (r0, r1)
        sems = (sem0, sem1)
        cid = lax.axis_index("c")
        sid = lax.axis_index("s")
        wid = cid * 16 + sid
        base = pl.multiple_of(sid * RPT, 8)
        pltpu.sync_copy(z_hbm.at[pl.ds(base, RPT)], acc_sh.at[pl.ds(base, RPT)])
        pltpu.sync_copy(src_hbm.at[wid], src_v)
        pltpu.sync_copy(dst_hbm.at[wid], dst_v)
        plsc.subcore_barrier()

        def block(b, carry):
            k0 = b * BLK
            d = [pltpu.async_copy(z_hbm.at[src_v.at[k0 + j]], rows[j % 2],
                                  sems[j % 2])
                 for j in range(2)]
            for j in range(BLK):
                d[j].wait()
                pltpu.sync_copy(rows[j % 2], acc_sh.at[dst_v.at[k0 + j]],
                                add=True)
                if j + 2 < BLK:
                    d.append(pltpu.async_copy(
                        z_hbm.at[src_v.at[k0 + j + 2]], rows[j % 2],
                        sems[j % 2]))
            return carry

        lax.fori_loop(0, NCH // BLK, block, 0)
        plsc.subcore_barrier()
        pltpu.sync_copy(acc_sh.at[pl.ds(base, RPT)],
                        out_hbm.at[cid, pl.ds(base, RPT)])

    return _agg


_agg128 = _make_agg(H)
_agg64 = _make_agg(D_OUT)


def _tc_pre_body(x_ref, deg_ref, wp_ref, bp_ref, w0_ref, h0_ref, z0_ref, dinv_ref):
    d = deg_ref[...]
    dinv = lax.rsqrt(d[:, 0:1] + d[:, 1:2] + 1.0)
    h0 = _lrelu(jnp.dot(x_ref[...], wp_ref[...],
                        preferred_element_type=jnp.float32) + bp_ref[...])
    h0_ref[...] = h0
    z0_ref[...] = jnp.dot(h0, w0_ref[...],
                          preferred_element_type=jnp.float32) * dinv
    dinv_ref[...] = dinv


_tc_pre = pl.pallas_call(
    _tc_pre_body,
    out_shape=(
        jax.ShapeDtypeStruct((NPAD, H), jnp.float32),
        jax.ShapeDtypeStruct((NPAD, H), jnp.float32),
        jax.ShapeDtypeStruct((NPAD, 1), jnp.float32),
    ),
)


def _make_tc_mid(dout):
    def body(acc_ref, z_ref, h_ref, dinv_ref, b_ref, g_ref, be_ref, w_ref,
             hn_ref, zn_ref):
        dinv = dinv_ref[...]
        s = (acc_ref[0] + acc_ref[1] - z_ref[...]) * dinv + b_ref[...]
        s = s * g_ref[...] + be_ref[...]
        hn = _lrelu(s) + h_ref[...]
        hn_ref[...] = hn
        zn_ref[...] = jnp.dot(hn, w_ref[...],
                              preferred_element_type=jnp.float32) * dinv

    return pl.pallas_call(
        body,
        out_shape=(
            jax.ShapeDtypeStruct((NPAD, H), jnp.float32),
            jax.ShapeDtypeStruct((NPAD, dout), jnp.float32),
        ),
    )


_tc_mid128 = _make_tc_mid(H)
_tc_mid64 = _make_tc_mid(D_OUT)


def _tc_fin_body(acc_ref, z_ref, dinv_ref, b2_ref, a1_ref, ab1_ref, a2_ref,
                 ab2_ref, wc_ref, out_ref):
    h3p = (acc_ref[0] + acc_ref[1] - z_ref[...]) * dinv_ref[...] + b2_ref[...]
    h3 = h3p[:N]
    a = _lrelu(jnp.dot(h3, a1_ref[...],
                       preferred_element_type=jnp.float32) + ab1_ref[...])
    s = jnp.dot(a, a2_ref[...], preferred_element_type=jnp.float32) + ab2_ref[...]
    m = jnp.max(s, axis=0, keepdims=True)
    e = jnp.exp(s - m)
    sm = e / jnp.sum(e, axis=0, keepdims=True)
    logit = jnp.sum(sm * wc_ref[...], axis=1, keepdims=True)
    cw = 1.0 / (1.0 + jnp.exp(-logit))
    out_ref[...] = h3 * cw


_tc_fin = pl.pallas_call(
    _tc_fin_body,
    out_shape=jax.ShapeDtypeStruct((N, D_OUT), jnp.float32),
)


def kernel(x, edge_index, Wp, bp, W0, b0, W1, b1, W2, b2, g0, be0, g1, be1,
           Aw1, Ab1, Aw2, Ab2, Wc):
    f32 = jnp.float32
    src3 = edge_index[0].astype(jnp.int32).reshape(NW, NCH, CB)
    dst3 = edge_index[1].astype(jnp.int32).reshape(NW, NCH, CB)
    zerosN = jnp.zeros((NPAD,), f32)
    xpad = jnp.pad(x, ((0, NPAD - N), (0, 0)))

    deg = _deg_kernel(dst3, zerosN)
    degT = deg.T

    h0, z0, dinv = _tc_pre(xpad, degT, Wp, bp.reshape(1, -1), W0)
    acc0 = _agg128(src3, dst3, z0)
    h1, z1 = _tc_mid128(acc0, z0, h0, dinv, b0.reshape(1, -1),
                        (g0 * _BN).reshape(1, -1), be0.reshape(1, -1), W1)
    acc1 = _agg128(src3, dst3, z1)
    h2, z2 = _tc_mid64(acc1, z1, h1, dinv, b1.reshape(1, -1),
                       (g1 * _BN).reshape(1, -1), be1.reshape(1, -1), W2)
    del h2
    acc2 = _agg64(src3, dst3, z2)

    a1cat = jnp.transpose(Aw1, (1, 0, 2)).reshape(D_OUT, D_OUT)
    ab1cat = Ab1.reshape(1, D_OUT)
    a2bd = jax.scipy.linalg.block_diag(*[Aw2[i] for i in range(4)])
    ab2cat = Ab2.reshape(1, 4)
    wcr = Wc.reshape(1, 4)

    return _tc_fin(acc2, z2, dinv, b2.reshape(1, -1), a1cat, ab1cat,
                   a2bd, ab2cat, wcr)


# speedup vs baseline: 3.6910x; 1.0224x over previous
import functools

import jax
import jax.numpy as jnp
from jax import lax
from jax.experimental import pallas as pl
from jax.experimental.pallas import tpu as pltpu
from jax.experimental.pallas import tpu_sc as plsc

N = 10000
D_IN = 128
H = 128
D_OUT = 64
E = 320000

NPAD = 10112
NW = 32
CB = 100
NCH = 100
BLK = 25
EPAD = NW * NCH * CB
RPT = NPAD // 16

_MESH = plsc.VectorSubcoreMesh(core_axis_name="c", subcore_axis_name="s")
_BN = float(1.0 / (1.0 + 1e-5) ** 0.5)


def _lrelu(t):
    return jnp.where(t >= 0, t, 0.2 * t)


@functools.partial(
    pl.kernel,
    out_type=jax.ShapeDtypeStruct((2, NPAD), jnp.float32),
    mesh=_MESH,
    scratch_types=[
        pltpu.VMEM((NCH, CB), jnp.int32),
        pltpu.VMEM((16 * (-(-CB // 16)),), jnp.float32),
        pltpu.VMEM_SHARED((NPAD,), jnp.float32),
    ],
    name="deg",
    compiler_params=pltpu.CompilerParams(use_tc_tiling_on_sc=False),
)
def _deg_kernel(eidx_hbm, zeros_hbm, out_hbm, dst_v, ones_v, deg_sh):
    cid = lax.axis_index("c")
    sid = lax.axis_index("s")
    wid = cid * 16 + sid
    base = pl.multiple_of(sid * RPT, 8)
    pltpu.sync_copy(zeros_hbm.at[pl.ds(base, RPT)], deg_sh.at[pl.ds(base, RPT)])
    pltpu.sync_copy(eidx_hbm.at[1, wid], dst_v)
    for i in range(-(-CB // 16)):
        ones_v[pl.ds(i * 16, 16)] = jnp.ones((16,), jnp.float32)
    plsc.subcore_barrier()

    def body(j, carry):
        pltpu.sync_copy(ones_v.at[pl.ds(0, CB)], deg_sh.at[dst_v.at[j]],
                        add=True)
        return carry

    lax.fori_loop(0, NCH, body, 0)
    plsc.subcore_barrier()
    pltpu.sync_copy(deg_sh.at[pl.ds(base, RPT)], out_hbm.at[cid, pl.ds(base, RPT)])


def _make_agg(hdim):
    @functools.partial(
        pl.kernel,
        out_type=jax.ShapeDtypeStruct((2, NPAD, hdim), jnp.float32),
        mesh=_MESH,
        scratch_types=[
            pltpu.VMEM((NCH, CB), jnp.int32),
            pltpu.VMEM((NCH, CB), jnp.int32),
            pltpu.VMEM((CB, hdim), jnp.float32),
            pltpu.VMEM((CB, hdim), jnp.float32),
            pltpu.VMEM_SHARED((NPAD, hdim), jnp.float32),
            pltpu.SemaphoreType.DMA,
            pltpu.SemaphoreType.DMA,
        ],
        name="agg%d" % hdim,
        compiler_params=pltpu.CompilerParams(use_tc_tiling_on_sc=False),
    )
    def _agg(eidx_hbm, z_hbm, out_hbm, src_v, dst_v, r0, r1, acc_sh,
             sem0, sem1):
        rows = (r0, r1)
        sems = (sem0, sem1)
        cid = lax.axis_index("c")
        sid = lax.axis_index("s")
        wid = cid * 16 + sid
        base = pl.multiple_of(sid * RPT, 8)
        pltpu.sync_copy(z_hbm.at[pl.ds(base, RPT)], acc_sh.at[pl.ds(base, RPT)])
        pltpu.sync_copy(eidx_hbm.at[0, wid], src_v)
        pltpu.sync_copy(eidx_hbm.at[1, wid], dst_v)
        plsc.subcore_barrier()

        def block(b, carry):
            k0 = b * BLK
            d = [pltpu.async_copy(z_hbm.at[src_v.at[k0 + j]], rows[j % 2],
                                  sems[j % 2])
                 for j in range(2)]
            for j in range(BLK):
                d[j].wait()
                pltpu.sync_copy(rows[j % 2], acc_sh.at[dst_v.at[k0 + j]],
                                add=True)
                if j + 2 < BLK:
                    d.append(pltpu.async_copy(
                        z_hbm.at[src_v.at[k0 + j + 2]], rows[j % 2],
                        sems[j % 2]))
            return carry

        lax.fori_loop(0, NCH // BLK, block, 0)
        plsc.subcore_barrier()
        pltpu.sync_copy(acc_sh.at[pl.ds(base, RPT)],
                        out_hbm.at[cid, pl.ds(base, RPT)])

    return _agg


_agg128 = _make_agg(H)
_agg64 = _make_agg(D_OUT)


def _tc_mm_body(x_ref, wp_ref, bp_ref, w0_ref, h0_ref, hw_ref):
    h0 = _lrelu(jnp.dot(x_ref[...], wp_ref[...],
                        preferred_element_type=jnp.float32) + bp_ref[...])
    zpad = jnp.zeros((NPAD - N, H), jnp.float32)
    h0_ref[...] = jnp.concatenate([h0, zpad], axis=0)
    hw = jnp.dot(h0, w0_ref[...], preferred_element_type=jnp.float32)
    hw_ref[...] = jnp.concatenate([hw, zpad], axis=0)


_tc_mm = pl.pallas_call(
    _tc_mm_body,
    out_shape=(
        jax.ShapeDtypeStruct((NPAD, H), jnp.float32),
        jax.ShapeDtypeStruct((NPAD, H), jnp.float32),
    ),
)


def _tc_z_body(hw_ref, deg_ref, z0_ref, dinv_ref):
    d = deg_ref[...]
    dinv = lax.rsqrt(d[:, 0:1] + d[:, 1:2] + 1.0)
    z0_ref[...] = hw_ref[...] * dinv
    dinv_ref[...] = dinv


_tc_z = pl.pallas_call(
    _tc_z_body,
    out_shape=(
        jax.ShapeDtypeStruct((NPAD, H), jnp.float32),
        jax.ShapeDtypeStruct((NPAD, 1), jnp.float32),
    ),
)


def _make_tc_mid(dout):
    def body(acc_ref, z_ref, h_ref, dinv_ref, b_ref, g_ref, be_ref, w_ref,
             hn_ref, zn_ref):
        dinv = dinv_ref[...]
        s = (acc_ref[0] + acc_ref[1] - z_ref[...]) * dinv + b_ref[...]
        s = s * g_ref[...] + be_ref[...]
        hn = _lrelu(s) + h_ref[...]
        hn_ref[...] = hn
        zn_ref[...] = jnp.dot(hn, w_ref[...],
                              preferred_element_type=jnp.float32) * dinv

    return pl.pallas_call(
        body,
        out_shape=(
            jax.ShapeDtypeStruct((NPAD, H), jnp.float32),
            jax.ShapeDtypeStruct((NPAD, dout), jnp.float32),
        ),
    )


_tc_mid128 = _make_tc_mid(H)
_tc_mid64 = _make_tc_mid(D_OUT)


def _tc_fin_body(acc_ref, z_ref, dinv_ref, b2_ref, a1_ref, ab1_ref, a2_ref,
                 ab2_ref, wc_ref, out_ref):
    h3p = (acc_ref[0] + acc_ref[1] - z_ref[...]) * dinv_ref[...] + b2_ref[...]
    h3 = h3p[:N]
    a = _lrelu(jnp.dot(h3, a1_ref[...],
                       preferred_element_type=jnp.float32) + ab1_ref[...])
    s = jnp.dot(a, a2_ref[...], preferred_element_type=jnp.float32) + ab2_ref[...]
    m = jnp.max(s, axis=0, keepdims=True)
    e = jnp.exp(s - m)
    sm = e / jnp.sum(e, axis=0, keepdims=True)
    logit = jnp.sum(sm * wc_ref[...], axis=1, keepdims=True)
    cw = 1.0 / (1.0 + jnp.exp(-logit))
    out_ref[...] = h3 * cw


_tc_fin = pl.pallas_call(
    _tc_fin_body,
    out_shape=jax.ShapeDtypeStruct((N, D_OUT), jnp.float32),
)


def kernel(x, edge_index, Wp, bp, W0, b0, W1, b1, W2, b2, g0, be0, g1, be1,
           Aw1, Ab1, Aw2, Ab2, Wc):
    f32 = jnp.float32
    eidx = edge_index.astype(jnp.int32).reshape(2, NW, NCH, CB)
    zerosN = jnp.zeros((NPAD,), f32)

    deg = _deg_kernel(eidx, zerosN)
    degT = deg.T

    h0, hw0 = _tc_mm(x, Wp, bp.reshape(1, -1), W0)
    z0, dinv = _tc_z(hw0, degT)
    acc0 = _agg128(eidx, z0)
    h1, z1 = _tc_mid128(acc0, z0, h0, dinv, b0.reshape(1, -1),
                        (g0 * _BN).reshape(1, -1), be0.reshape(1, -1), W1)
    acc1 = _agg128(eidx, z1)
    h2, z2 = _tc_mid64(acc1, z1, h1, dinv, b1.reshape(1, -1),
                       (g1 * _BN).reshape(1, -1), be1.reshape(1, -1), W2)
    del h2
    acc2 = _agg64(eidx, z2)

    a1cat = jnp.transpose(Aw1, (1, 0, 2)).reshape(D_OUT, D_OUT)
    ab1cat = Ab1.reshape(1, D_OUT)
    a2bd = jax.scipy.linalg.block_diag(*[Aw2[i] for i in range(4)])
    ab2cat = Ab2.reshape(1, 4)
    wcr = Wc.reshape(1, 4)

    return _tc_fin(acc2, z2, dinv, b2.reshape(1, -1), a1cat, ab1cat,
                   a2bd, ab2cat, wcr)


# speedup vs baseline: 3.8796x; 1.0511x over previous
import functools

import jax
import jax.numpy as jnp
from jax import lax
from jax.experimental import pallas as pl
from jax.experimental.pallas import tpu as pltpu
from jax.experimental.pallas import tpu_sc as plsc

N = 10000
D_IN = 128
H = 128
D_OUT = 64
E = 320000

NPAD = 10112
NW = 32
CB = 100
NCH = 100
BLK = 25
EPAD = NW * NCH * CB
RPT = NPAD // 16

_MESH = plsc.VectorSubcoreMesh(core_axis_name="c", subcore_axis_name="s")
_BN = float(1.0 / (1.0 + 1e-5) ** 0.5)


def _lrelu(t):
    return jnp.where(t >= 0, t, 0.2 * t)


@functools.partial(
    pl.kernel,
    out_type=jax.ShapeDtypeStruct((2, NPAD), jnp.float32),
    mesh=_MESH,
    scratch_types=[
        pltpu.VMEM((NCH, CB), jnp.int32),
        pltpu.VMEM((16 * (-(-CB // 16)),), jnp.float32),
        pltpu.VMEM_SHARED((NPAD,), jnp.float32),
    ],
    name="deg",
    compiler_params=pltpu.CompilerParams(use_tc_tiling_on_sc=False),
)
def _deg_kernel(eidx_hbm, zeros_hbm, out_hbm, dst_v, ones_v, deg_sh):
    cid = lax.axis_index("c")
    sid = lax.axis_index("s")
    wid = cid * 16 + sid
    base = pl.multiple_of(sid * RPT, 8)
    pltpu.sync_copy(zeros_hbm.at[pl.ds(base, RPT)], deg_sh.at[pl.ds(base, RPT)])
    pltpu.sync_copy(eidx_hbm.at[1, wid], dst_v)
    for i in range(-(-CB // 16)):
        ones_v[pl.ds(i * 16, 16)] = jnp.ones((16,), jnp.float32)
    plsc.subcore_barrier()

    def body(j, carry):
        pltpu.sync_copy(ones_v.at[pl.ds(0, CB)], deg_sh.at[dst_v.at[j]],
                        add=True)
        return carry

    lax.fori_loop(0, NCH, body, 0)
    plsc.subcore_barrier()
    pltpu.sync_copy(deg_sh.at[pl.ds(base, RPT)], out_hbm.at[cid, pl.ds(base, RPT)])


def _make_agg(hdim, nbuf):
    @functools.partial(
        pl.kernel,
        out_type=jax.ShapeDtypeStruct((2, NPAD, hdim), jnp.float32),
        mesh=_MESH,
        scratch_types=(
            [pltpu.VMEM((NCH, CB), jnp.int32)] * 2
            + [pltpu.VMEM((CB, hdim), jnp.float32) for _ in range(nbuf)]
            + [pltpu.VMEM_SHARED((NPAD, hdim), jnp.float32)]
            + [pltpu.SemaphoreType.DMA for _ in range(nbuf + 1)]
        ),
        name="agg%d" % hdim,
        compiler_params=pltpu.CompilerParams(use_tc_tiling_on_sc=False),
    )
    def _agg(eidx_hbm, z_hbm, out_hbm, src_v, dst_v, *rest):
        rows = rest[:nbuf]
        acc_sh = rest[nbuf]
        sems = rest[nbuf + 1:2 * nbuf + 1]
        isem = rest[2 * nbuf + 1]
        cid = lax.axis_index("c")
        sid = lax.axis_index("s")
        wid = cid * 16 + sid
        base = pl.multiple_of(sid * RPT, 8)
        pltpu.sync_copy(eidx_hbm.at[0, wid], src_v)
        pltpu.sync_copy(eidx_hbm.at[1, wid], dst_v)
        ini = pltpu.async_copy(z_hbm.at[pl.ds(base, RPT)],
                               acc_sh.at[pl.ds(base, RPT)], isem)
        primed = [pltpu.async_copy(z_hbm.at[src_v.at[j]], rows[j % nbuf],
                                   sems[j % nbuf])
                  for j in range(nbuf)]
        ini.wait()
        plsc.subcore_barrier()

        def run_block(k0, d):
            for j in range(BLK):
                d[j].wait()
                pltpu.sync_copy(rows[j % nbuf], acc_sh.at[dst_v.at[k0 + j]],
                                add=True)
                if j + nbuf < BLK:
                    d.append(pltpu.async_copy(
                        z_hbm.at[src_v.at[k0 + j + nbuf]], rows[j % nbuf],
                        sems[j % nbuf]))

        run_block(0, primed)

        def block(b, carry):
            k0 = b * BLK
            run_block(k0, [pltpu.async_copy(z_hbm.at[src_v.at[k0 + j]],
                                            rows[j % nbuf], sems[j % nbuf])
                           for j in range(nbuf)])
            return carry

        lax.fori_loop(1, NCH // BLK, block, 0)
        plsc.subcore_barrier()
        pltpu.sync_copy(acc_sh.at[pl.ds(base, RPT)],
                        out_hbm.at[cid, pl.ds(base, RPT)])

    return _agg


_agg128 = _make_agg(H, 2)
_agg64 = _make_agg(D_OUT, 3)


def _tc_mm_body(x_ref, wp_ref, bp_ref, w0_ref, h0_ref, hw_ref):
    h0 = _lrelu(jnp.dot(x_ref[...], wp_ref[...],
                        preferred_element_type=jnp.float32) + bp_ref[...])
    zpad = jnp.zeros((NPAD - N, H), jnp.float32)
    h0_ref[...] = jnp.concatenate([h0, zpad], axis=0)
    hw = jnp.dot(h0, w0_ref[...], preferred_element_type=jnp.float32)
    hw_ref[...] = jnp.concatenate([hw, zpad], axis=0)


_tc_mm = pl.pallas_call(
    _tc_mm_body,
    out_shape=(
        jax.ShapeDtypeStruct((NPAD, H), jnp.float32),
        jax.ShapeDtypeStruct((NPAD, H), jnp.float32),
    ),
)


def _tc_z_body(hw_ref, deg_ref, z0_ref, dinv_ref):
    d = deg_ref[...]
    dinv = lax.rsqrt(d[:, 0:1] + d[:, 1:2] + 1.0)
    z0_ref[...] = hw_ref[...] * dinv
    dinv_ref[...] = dinv


_tc_z = pl.pallas_call(
    _tc_z_body,
    out_shape=(
        jax.ShapeDtypeStruct((NPAD, H), jnp.float32),
        jax.ShapeDtypeStruct((NPAD, 1), jnp.float32),
    ),
)


def _make_tc_mid(dout):
    def body(acc_ref, z_ref, h_ref, dinv_ref, b_ref, g_ref, be_ref, w_ref,
             hn_ref, zn_ref):
        dinv = dinv_ref[...]
        s = (acc_ref[0] + acc_ref[1] - z_ref[...]) * dinv + b_ref[...]
        s = s * g_ref[...] + be_ref[...]
        hn = _lrelu(s) + h_ref[...]
        hn_ref[...] = hn
        zn_ref[...] = jnp.dot(hn, w_ref[...],
                              preferred_element_type=jnp.float32) * dinv

    return pl.pallas_call(
        body,
        out_shape=(
            jax.ShapeDtypeStruct((NPAD, H), jnp.float32),
            jax.ShapeDtypeStruct((NPAD, dout), jnp.float32),
        ),
    )


_tc_mid128 = _make_tc_mid(H)
_tc_mid64 = _make_tc_mid(D_OUT)


def _tc_fin_body(acc_ref, z_ref, dinv_ref, b2_ref, a1_ref, ab1_ref, a2_ref,
                 ab2_ref, wc_ref, out_ref):
    h3p = (acc_ref[0] + acc_ref[1] - z_ref[...]) * dinv_ref[...] + b2_ref[...]
    h3 = h3p[:N]
    a = _lrelu(jnp.dot(h3, a1_ref[...],
                       preferred_element_type=jnp.float32) + ab1_ref[...])
    s = jnp.dot(a, a2_ref[...], preferred_element_type=jnp.float32) + ab2_ref[...]
    m = jnp.max(s, axis=0, keepdims=True)
    e = jnp.exp(s - m)
    sm = e / jnp.sum(e, axis=0, keepdims=True)
    logit = jnp.sum(sm * wc_ref[...], axis=1, keepdims=True)
    cw = 1.0 / (1.0 + jnp.exp(-logit))
    out_ref[...] = h3 * cw


_tc_fin = pl.pallas_call(
    _tc_fin_body,
    out_shape=jax.ShapeDtypeStruct((N, D_OUT), jnp.float32),
)


def kernel(x, edge_index, Wp, bp, W0, b0, W1, b1, W2, b2, g0, be0, g1, be1,
           Aw1, Ab1, Aw2, Ab2, Wc):
    f32 = jnp.float32
    eidx = edge_index.astype(jnp.int32).reshape(2, NW, NCH, CB)
    zerosN = jnp.zeros((NPAD,), f32)

    deg = _deg_kernel(eidx, zerosN)
    degT = deg.T

    h0, hw0 = _tc_mm(x, Wp, bp.reshape(1, -1), W0)
    z0, dinv = _tc_z(hw0, degT)
    acc0 = _agg128(eidx, z0)
    h1, z1 = _tc_mid128(acc0, z0, h0, dinv, b0.reshape(1, -1),
                        (g0 * _BN).reshape(1, -1), be0.reshape(1, -1), W1)
    acc1 = _agg128(eidx, z1)
    h2, z2 = _tc_mid64(acc1, z1, h1, dinv, b1.reshape(1, -1),
                       (g1 * _BN).reshape(1, -1), be1.reshape(1, -1), W2)
    del h2
    acc2 = _agg64(eidx, z2)

    a1cat = jnp.transpose(Aw1, (1, 0, 2)).reshape(D_OUT, D_OUT)
    ab1cat = Ab1.reshape(1, D_OUT)
    a2bd = jax.scipy.linalg.block_diag(*[Aw2[i] for i in range(4)])
    ab2cat = Ab2.reshape(1, 4)
    wcr = Wc.reshape(1, 4)

    return _tc_fin(acc2, z2, dinv, b2.reshape(1, -1), a1cat, ab1cat,
                   a2bd, ab2cat, wcr)


# speedup vs baseline: 3.8998x; 1.0052x over previous
import functools

import jax
import jax.numpy as jnp
from jax import lax
from jax.experimental import pallas as pl
from jax.experimental.pallas import tpu as pltpu
from jax.experimental.pallas import tpu_sc as plsc

N = 10000
D_IN = 128
H = 128
D_OUT = 64
E = 320000

NPAD = 10112
NW = 32
CB = 100
NCH = 100
BLK = 100
EPAD = NW * NCH * CB
RPT = NPAD // 16

_MESH = plsc.VectorSubcoreMesh(core_axis_name="c", subcore_axis_name="s")
_BN = float(1.0 / (1.0 + 1e-5) ** 0.5)


def _lrelu(t):
    return jnp.where(t >= 0, t, 0.2 * t)


@functools.partial(
    pl.kernel,
    out_type=jax.ShapeDtypeStruct((2, NPAD), jnp.float32),
    mesh=_MESH,
    scratch_types=[
        pltpu.VMEM((NCH, CB), jnp.int32),
        pltpu.VMEM((16 * (-(-CB // 16)),), jnp.float32),
        pltpu.VMEM_SHARED((NPAD,), jnp.float32),
    ],
    name="deg",
    compiler_params=pltpu.CompilerParams(use_tc_tiling_on_sc=False),
)
def _deg_kernel(eidx_hbm, zeros_hbm, out_hbm, dst_v, ones_v, deg_sh):
    cid = lax.axis_index("c")
    sid = lax.axis_index("s")
    wid = cid * 16 + sid
    base = pl.multiple_of(sid * RPT, 8)
    pltpu.sync_copy(zeros_hbm.at[pl.ds(base, RPT)], deg_sh.at[pl.ds(base, RPT)])
    pltpu.sync_copy(eidx_hbm.at[1, wid], dst_v)
    for i in range(-(-CB // 16)):
        ones_v[pl.ds(i * 16, 16)] = jnp.ones((16,), jnp.float32)
    plsc.subcore_barrier()

    def body(j, carry):
        pltpu.sync_copy(ones_v.at[pl.ds(0, CB)], deg_sh.at[dst_v.at[j]],
                        add=True)
        return carry

    lax.fori_loop(0, NCH, body, 0)
    plsc.subcore_barrier()
    pltpu.sync_copy(deg_sh.at[pl.ds(base, RPT)], out_hbm.at[cid, pl.ds(base, RPT)])


def _make_agg(hdim, nbuf):
    @functools.partial(
        pl.kernel,
        out_type=jax.ShapeDtypeStruct((2, NPAD, hdim), jnp.float32),
        mesh=_MESH,
        scratch_types=(
            [pltpu.VMEM((NCH, CB), jnp.int32)] * 2
            + [pltpu.VMEM((CB, hdim), jnp.float32) for _ in range(nbuf)]
            + [pltpu.VMEM_SHARED((NPAD, hdim), jnp.float32)]
            + [pltpu.SemaphoreType.DMA for _ in range(nbuf + 1)]
        ),
        name="agg%d" % hdim,
        compiler_params=pltpu.CompilerParams(use_tc_tiling_on_sc=False),
    )
    def _agg(eidx_hbm, z_hbm, out_hbm, src_v, dst_v, *rest):
        rows = rest[:nbuf]
        acc_sh = rest[nbuf]
        sems = rest[nbuf + 1:2 * nbuf + 1]
        isem = rest[2 * nbuf + 1]
        cid = lax.axis_index("c")
        sid = lax.axis_index("s")
        wid = cid * 16 + sid
        base = pl.multiple_of(sid * RPT, 8)
        pltpu.sync_copy(eidx_hbm.at[0, wid], src_v)
        pltpu.sync_copy(eidx_hbm.at[1, wid], dst_v)
        ini = pltpu.async_copy(z_hbm.at[pl.ds(base, RPT)],
                               acc_sh.at[pl.ds(base, RPT)], isem)
        primed = [pltpu.async_copy(z_hbm.at[src_v.at[j]], rows[j % nbuf],
                                   sems[j % nbuf])
                  for j in range(nbuf)]
        ini.wait()
        plsc.subcore_barrier()

        def run_block(k0, d):
            for j in range(BLK):
                d[j].wait()
                pltpu.sync_copy(rows[j % nbuf], acc_sh.at[dst_v.at[k0 + j]],
                                add=True)
                if j + nbuf < BLK:
                    d.append(pltpu.async_copy(
                        z_hbm.at[src_v.at[k0 + j + nbuf]], rows[j % nbuf],
                        sems[j % nbuf]))

        run_block(0, primed)

        def block(b, carry):
            k0 = b * BLK
            run_block(k0, [pltpu.async_copy(z_hbm.at[src_v.at[k0 + j]],
                                            rows[j % nbuf], sems[j % nbuf])
                           for j in range(nbuf)])
            return carry

        lax.fori_loop(1, NCH // BLK, block, 0)
        plsc.subcore_barrier()
        pltpu.sync_copy(acc_sh.at[pl.ds(base, RPT)],
                        out_hbm.at[cid, pl.ds(base, RPT)])

    return _agg


_agg128 = _make_agg(H, 2)
_agg64 = _make_agg(D_OUT, 3)


def _tc_mm_body(x_ref, wp_ref, bp_ref, w0_ref, h0_ref, hw_ref):
    h0 = _lrelu(jnp.dot(x_ref[...], wp_ref[...],
                        preferred_element_type=jnp.float32) + bp_ref[...])
    zpad = jnp.zeros((NPAD - N, H), jnp.float32)
    h0_ref[...] = jnp.concatenate([h0, zpad], axis=0)
    hw = jnp.dot(h0, w0_ref[...], preferred_element_type=jnp.float32)
    hw_ref[...] = jnp.concatenate([hw, zpad], axis=0)


_tc_mm = pl.pallas_call(
    _tc_mm_body,
    out_shape=(
        jax.ShapeDtypeStruct((NPAD, H), jnp.float32),
        jax.ShapeDtypeStruct((NPAD, H), jnp.float32),
    ),
)


def _tc_z_body(hw_ref, deg_ref, z0_ref, dinv_ref):
    d = deg_ref[...]
    dinv = lax.rsqrt(d[:, 0:1] + d[:, 1:2] + 1.0)
    z0_ref[...] = hw_ref[...] * dinv
    dinv_ref[...] = dinv


_tc_z = pl.pallas_call(
    _tc_z_body,
    out_shape=(
        jax.ShapeDtypeStruct((NPAD, H), jnp.float32),
        jax.ShapeDtypeStruct((NPAD, 1), jnp.float32),
    ),
)


def _make_tc_mid(dout):
    def body(acc_ref, z_ref, h_ref, dinv_ref, b_ref, g_ref, be_ref, w_ref,
             hn_ref, zn_ref):
        dinv = dinv_ref[...]
        s = (acc_ref[0] + acc_ref[1] - z_ref[...]) * dinv + b_ref[...]
        s = s * g_ref[...] + be_ref[...]
        hn = _lrelu(s) + h_ref[...]
        hn_ref[...] = hn
        zn_ref[...] = jnp.dot(hn, w_ref[...],
                              preferred_element_type=jnp.float32) * dinv

    return pl.pallas_call(
        body,
        out_shape=(
            jax.ShapeDtypeStruct((NPAD, H), jnp.float32),
            jax.ShapeDtypeStruct((NPAD, dout), jnp.float32),
        ),
    )


_tc_mid128 = _make_tc_mid(H)
_tc_mid64 = _make_tc_mid(D_OUT)


def _tc_fin_body(acc_ref, z_ref, dinv_ref, b2_ref, a1_ref, ab1_ref, a2_ref,
                 ab2_ref, wc_ref, out_ref):
    h3p = (acc_ref[0] + acc_ref[1] - z_ref[...]) * dinv_ref[...] + b2_ref[...]
    h3 = h3p[:N]
    a = _lrelu(jnp.dot(h3, a1_ref[...],
                       preferred_element_type=jnp.float32) + ab1_ref[...])
    s = jnp.dot(a, a2_ref[...], preferred_element_type=jnp.float32) + ab2_ref[...]
    m = jnp.max(s, axis=0, keepdims=True)
    e = jnp.exp(s - m)
    sm = e / jnp.sum(e, axis=0, keepdims=True)
    logit = jnp.sum(sm * wc_ref[...], axis=1, keepdims=True)
    cw = 1.0 / (1.0 + jnp.exp(-logit))
    out_ref[...] = h3 * cw


_tc_fin = pl.pallas_call(
    _tc_fin_body,
    out_shape=jax.ShapeDtypeStruct((N, D_OUT), jnp.float32),
)


def kernel(x, edge_index, Wp, bp, W0, b0, W1, b1, W2, b2, g0, be0, g1, be1,
           Aw1, Ab1, Aw2, Ab2, Wc):
    f32 = jnp.float32
    eidx = edge_index.astype(jnp.int32).reshape(2, NW, NCH, CB)
    zerosN = jnp.zeros((NPAD,), f32)

    deg = _deg_kernel(eidx, zerosN)
    degT = deg.T

    h0, hw0 = _tc_mm(x, Wp, bp.reshape(1, -1), W0)
    z0, dinv = _tc_z(hw0, degT)
    acc0 = _agg128(eidx, z0)
    h1, z1 = _tc_mid128(acc0, z0, h0, dinv, b0.reshape(1, -1),
                        (g0 * _BN).reshape(1, -1), be0.reshape(1, -1), W1)
    acc1 = _agg128(eidx, z1)
    h2, z2 = _tc_mid64(acc1, z1, h1, dinv, b1.reshape(1, -1),
                       (g1 * _BN).reshape(1, -1), be1.reshape(1, -1), W2)
    del h2
    acc2 = _agg64(eidx, z2)

    a1cat = jnp.transpose(Aw1, (1, 0, 2)).reshape(D_OUT, D_OUT)
    ab1cat = Ab1.reshape(1, D_OUT)
    a2bd = jax.scipy.linalg.block_diag(*[Aw2[i] for i in range(4)])
    ab2cat = Ab2.reshape(1, 4)
    wcr = Wc.reshape(1, 4)

    return _tc_fin(acc2, z2, dinv, b2.reshape(1, -1), a1cat, ab1cat,
                   a2bd, ab2cat, wcr)
